# Initial kernel scaffold; baseline (speedup 1.0000x reference)
import jax
import jax.numpy as jnp
from jax.experimental import pallas as pl


def kernel(xyz, points, W0, c0, g0, b0, W1, c1, g1, b1, W2, c2, g2, b2):
    raise NotImplementedError("write your pallas kernel here")


# speedup vs baseline: 3.0790x; 3.0790x over previous
import functools

import jax
import jax.numpy as jnp
from jax.experimental import pallas as pl
from jax.experimental.pallas import tpu as pltpu

NPOINT = 512
RADIUS = 0.2
NSAMPLE = 32
B = 16
N = 4096
SCHUNK = 128
NCHUNKS = NPOINT // SCHUNK
HIGH = jax.lax.Precision.HIGHEST


def _dot_bf16(a, b, dims):
    return jax.lax.dot_general(a.astype(jnp.bfloat16), b.astype(jnp.bfloat16),
                               dims, preferred_element_type=jnp.float32)


def _fps_kernel(xyz_ref, out_ref):
    x = xyz_ref[:, 0]
    y = xyz_ref[:, 1]
    z = xyz_ref[:, 2]
    gidx = (jax.lax.broadcasted_iota(jnp.int32, (B, 32, 128), 1) * 128
            + jax.lax.broadcasted_iota(jnp.int32, (B, 32, 128), 2))
    ridx = (jax.lax.broadcasted_iota(jnp.int32, (B, 4, 128), 1) * 128
            + jax.lax.broadcasted_iota(jnp.int32, (B, 4, 128), 2))

    def red2(v, fn):
        return fn(fn(v, axis=2, keepdims=True), axis=1, keepdims=True)

    def body(i, carry):
        dist, far, ax, ay, az = carry
        sel = (gidx == far).astype(jnp.float32)
        cx = red2(sel * x, jnp.sum)
        cy = red2(sel * y, jnp.sum)
        cz = red2(sel * z, jnp.sum)
        m512 = ridx == i
        ax = jnp.where(m512, cx, ax)
        ay = jnp.where(m512, cy, ay)
        az = jnp.where(m512, cz, az)
        dx = x - cx
        dy = y - cy
        dz = z - cz
        d = (dx * dx + dy * dy) + dz * dz
        dist = jnp.where(d < dist, d, dist)
        mx = red2(dist, jnp.max)
        far = red2(jnp.where(dist == mx, gidx, N), jnp.min)
        return dist, far, ax, ay, az

    dist0 = jnp.full((B, 32, 128), 1e10, dtype=jnp.float32)
    far0 = jnp.zeros((B, 1, 1), dtype=jnp.int32)
    acc0 = jnp.zeros((B, 4, 128), dtype=jnp.float32)
    _, _, ax, ay, az = jax.lax.fori_loop(
        0, NPOINT, body, (dist0, far0, acc0, acc0, acc0))
    out_ref[:, 0] = ax
    out_ref[:, 1] = ay
    out_ref[:, 2] = az


def _group_kernel(feat_ref, nxyz_ref, w0_ref, c0_ref, z1_ref, st_ref):
    f = feat_ref[0]
    nc = nxyz_ref[0]
    xyz3 = f[0:3]
    sqx = (f[0:1] * f[0:1] + f[1:2] * f[1:2]) + f[2:3] * f[2:3]
    nx = nc[:, 0:1]
    ny = nc[:, 1:2]
    nz = nc[:, 2:3]
    sqs = (nx * nx + ny * ny) + nz * nz
    inner = _dot_bf16(nc, xyz3, (((1,), (0,)), ((), ())))
    sqrd = (sqs + sqx) - 2.0 * inner
    msk = (sqrd <= RADIUS * RADIUS).astype(jnp.float32)

    tri = (jax.lax.broadcasted_iota(jnp.int32, (128, 128), 0)
           <= jax.lax.broadcasted_iota(jnp.int32, (128, 128), 1)
           ).astype(jnp.float32)
    offset = jnp.zeros((SCHUNK, 1), dtype=jnp.float32)
    cparts = []
    for k in range(N // 128):
        mk = msk[:, k * 128:(k + 1) * 128]
        csk = jax.lax.dot_general(mk, tri, (((1,), (0,)), ((), ())),
                                  precision=HIGH)
        cparts.append(csk + offset)
        offset = offset + csk[:, 127:128]
    c = jnp.concatenate(cparts, axis=1)
    cc = msk * c
    count = offset

    w0 = w0_ref[...]
    c0 = c0_ref[...]
    cvec = jnp.concatenate(
        [nc, jnp.zeros((SCHUNK, 3), dtype=jnp.float32)], axis=1)

    s_acc = jnp.zeros((64, 1), dtype=jnp.float32)
    q_acc = jnp.zeros((64, 1), dtype=jnp.float32)
    g_first = None
    for j in range(NSAMPLE):
        match = (cc == float(j + 1)).astype(jnp.float32)
        g = jax.lax.dot_general(match, f, (((1,), (1,)), ((), ())),
                                precision=HIGH)
        gc = g - cvec
        if j == 0:
            g_first = gc
            gfin = gc
        else:
            gfin = jnp.where(count >= float(j + 1), gc, g_first)
        z = _dot_bf16(w0, gfin, (((1,), (1,)), ((), ()))) + c0
        z1_ref[0, j] = z
        s_acc = s_acc + jnp.sum(z, axis=1, keepdims=True)
        q_acc = q_acc + jnp.sum(z * z, axis=1, keepdims=True)

    stacked = jnp.concatenate(
        [s_acc, q_acc, jnp.zeros((64, 6), dtype=jnp.float32)], axis=1)
    prev = jnp.where(pl.program_id(1) == 0,
                     jnp.zeros((64, 8), dtype=jnp.float32), st_ref[0])
    st_ref[0] = prev + stacked


def _mlp_kernel(z_ref, sc_ref, bi_ref, w_ref, cb_ref, out_ref, st_ref, *,
                cout):
    sc = sc_ref[...]
    bi = bi_ref[...]
    w = w_ref[...]
    cb = cb_ref[...]
    s_acc = jnp.zeros((cout, 1), dtype=jnp.float32)
    q_acc = jnp.zeros((cout, 1), dtype=jnp.float32)
    for j in range(NSAMPLE):
        h = jnp.maximum(z_ref[0, j] * sc + bi, 0.0)
        z = _dot_bf16(w, h, (((1,), (0,)), ((), ()))) + cb
        out_ref[0, j] = z
        s_acc = s_acc + jnp.sum(z, axis=1, keepdims=True)
        q_acc = q_acc + jnp.sum(z * z, axis=1, keepdims=True)
    st_ref[0] = jnp.concatenate(
        [s_acc, q_acc, jnp.zeros((cout, 6), dtype=jnp.float32)], axis=1)


def _pool_kernel(z_ref, sc_ref, bi_ref, out_ref):
    sc = sc_ref[...]
    bi = bi_ref[...]
    m = jnp.maximum(z_ref[0, 0] * sc + bi, 0.0)
    for j in range(1, NSAMPLE):
        m = jnp.maximum(m, jnp.maximum(z_ref[0, j] * sc + bi, 0.0))
    out_ref[0] = m


def _bn_coeffs(stats, gamma, beta):
    cnt = float(B * NPOINT * NSAMPLE)
    s = jnp.sum(stats[:, :, 0], axis=0)
    q = jnp.sum(stats[:, :, 1], axis=0)
    mean = s / cnt
    var = q / cnt - mean * mean
    rstd = 1.0 / jnp.sqrt(var + 1e-5)
    scale = gamma * rstd
    bias = beta - mean * scale
    return scale[:, None], bias[:, None]


@jax.jit
def kernel(xyz, points, W0, c0, g0, b0, W1, c1, g1, b1, W2, c2, g2, b2):
    xyz4 = xyz.reshape(B, 3, 32, 128)
    nxyz = pl.pallas_call(
        _fps_kernel,
        out_shape=jax.ShapeDtypeStruct((B, 3, 4, 128), jnp.float32),
    )(xyz4)
    new_xyz_out = nxyz.reshape(B, 3, NPOINT)

    feat = jnp.concatenate([xyz, points], axis=1)
    nct = jnp.transpose(new_xyz_out, (0, 2, 1))
    w0m = W0[:, :, 0, 0]

    z1, st1 = pl.pallas_call(
        _group_kernel,
        grid=(B, NCHUNKS),
        in_specs=[
            pl.BlockSpec((1, 6, N), lambda b, s: (b, 0, 0)),
            pl.BlockSpec((1, SCHUNK, 3), lambda b, s: (b, s, 0)),
            pl.BlockSpec((64, 6), lambda b, s: (0, 0)),
            pl.BlockSpec((64, 1), lambda b, s: (0, 0)),
        ],
        out_specs=[
            pl.BlockSpec((1, NSAMPLE, 64, SCHUNK), lambda b, s: (b, 0, 0, s)),
            pl.BlockSpec((1, 64, 8), lambda b, s: (b, 0, 0)),
        ],
        out_shape=[
            jax.ShapeDtypeStruct((B, NSAMPLE, 64, NPOINT), jnp.float32),
            jax.ShapeDtypeStruct((B, 64, 8), jnp.float32),
        ],
        compiler_params=pltpu.CompilerParams(
            dimension_semantics=("parallel", "arbitrary")),
    )(feat, nct, w0m, c0[:, None])

    sc1, bi1 = _bn_coeffs(st1, g0, b0)
    z2, st2 = pl.pallas_call(
        functools.partial(_mlp_kernel, cout=64),
        grid=(B,),
        in_specs=[
            pl.BlockSpec((1, NSAMPLE, 64, NPOINT), lambda b: (b, 0, 0, 0)),
            pl.BlockSpec((64, 1), lambda b: (0, 0)),
            pl.BlockSpec((64, 1), lambda b: (0, 0)),
            pl.BlockSpec((64, 64), lambda b: (0, 0)),
            pl.BlockSpec((64, 1), lambda b: (0, 0)),
        ],
        out_specs=[
            pl.BlockSpec((1, NSAMPLE, 64, NPOINT), lambda b: (b, 0, 0, 0)),
            pl.BlockSpec((1, 64, 8), lambda b: (b, 0, 0)),
        ],
        out_shape=[
            jax.ShapeDtypeStruct((B, NSAMPLE, 64, NPOINT), jnp.float32),
            jax.ShapeDtypeStruct((B, 64, 8), jnp.float32),
        ],
        compiler_params=pltpu.CompilerParams(
            dimension_semantics=("parallel",)),
    )(z1, sc1, bi1, W1[:, :, 0, 0], c1[:, None])

    sc2, bi2 = _bn_coeffs(st2, g1, b1)
    z3, st3 = pl.pallas_call(
        functools.partial(_mlp_kernel, cout=128),
        grid=(B,),
        in_specs=[
            pl.BlockSpec((1, NSAMPLE, 64, NPOINT), lambda b: (b, 0, 0, 0)),
            pl.BlockSpec((64, 1), lambda b: (0, 0)),
            pl.BlockSpec((64, 1), lambda b: (0, 0)),
            pl.BlockSpec((128, 64), lambda b: (0, 0)),
            pl.BlockSpec((128, 1), lambda b: (0, 0)),
        ],
        out_specs=[
            pl.BlockSpec((1, NSAMPLE, 128, NPOINT), lambda b: (b, 0, 0, 0)),
            pl.BlockSpec((1, 128, 8), lambda b: (b, 0, 0)),
        ],
        out_shape=[
            jax.ShapeDtypeStruct((B, NSAMPLE, 128, NPOINT), jnp.float32),
            jax.ShapeDtypeStruct((B, 128, 8), jnp.float32),
        ],
        compiler_params=pltpu.CompilerParams(
            dimension_semantics=("parallel",)),
    )(z2, sc2, bi2, W2[:, :, 0, 0], c2[:, None])

    sc3, bi3 = _bn_coeffs(st3, g2, b2)
    new_points_out = pl.pallas_call(
        _pool_kernel,
        grid=(B,),
        in_specs=[
            pl.BlockSpec((1, NSAMPLE, 128, NPOINT), lambda b: (b, 0, 0, 0)),
            pl.BlockSpec((128, 1), lambda b: (0, 0)),
            pl.BlockSpec((128, 1), lambda b: (0, 0)),
        ],
        out_specs=pl.BlockSpec((1, 128, NPOINT), lambda b: (b, 0, 0)),
        out_shape=jax.ShapeDtypeStruct((B, 128, NPOINT), jnp.float32),
        compiler_params=pltpu.CompilerParams(
            dimension_semantics=("parallel",)),
    )(z3, sc3, bi3)

    return (new_xyz_out, new_points_out)


# speedup vs baseline: 7.9978x; 2.5975x over previous
import functools

import jax
import jax.numpy as jnp
from jax.experimental import pallas as pl
from jax.experimental.pallas import tpu as pltpu

NPOINT = 512
RADIUS = 0.2
NSAMPLE = 32
B = 16
N = 4096
SCHUNK = 128
NCHUNKS = NPOINT // SCHUNK
HIGH = jax.lax.Precision.HIGHEST


def _dot_bf16(a, b, dims):
    return jax.lax.dot_general(a.astype(jnp.bfloat16), b.astype(jnp.bfloat16),
                               dims, preferred_element_type=jnp.float32)


def _fps_kernel(xyz_ref, out_ref):
    x = xyz_ref[:, 0]
    y = xyz_ref[:, 1]
    z = xyz_ref[:, 2]
    gidx = (jax.lax.broadcasted_iota(jnp.int32, (B, 32, 128), 1) * 128
            + jax.lax.broadcasted_iota(jnp.int32, (B, 32, 128), 2))
    ridx = (jax.lax.broadcasted_iota(jnp.int32, (B, 4, 128), 1) * 128
            + jax.lax.broadcasted_iota(jnp.int32, (B, 4, 128), 2))

    def red2(v, fn):
        return fn(fn(v, axis=2, keepdims=True), axis=1, keepdims=True)

    def body(i, carry):
        dist, far, ax, ay, az = carry
        sel = (gidx == far).astype(jnp.float32)
        cx = red2(sel * x, jnp.sum)
        cy = red2(sel * y, jnp.sum)
        cz = red2(sel * z, jnp.sum)
        m512 = ridx == i
        ax = jnp.where(m512, cx, ax)
        ay = jnp.where(m512, cy, ay)
        az = jnp.where(m512, cz, az)
        dx = x - cx
        dy = y - cy
        dz = z - cz
        d = (dx * dx + dy * dy) + dz * dz
        dist = jnp.where(d < dist, d, dist)
        mx = red2(dist, jnp.max)
        far = red2(jnp.where(dist == mx, gidx, N), jnp.min)
        return dist, far, ax, ay, az

    dist0 = jnp.full((B, 32, 128), 1e10, dtype=jnp.float32)
    far0 = jnp.zeros((B, 1, 1), dtype=jnp.int32)
    acc0 = jnp.zeros((B, 4, 128), dtype=jnp.float32)
    _, _, ax, ay, az = jax.lax.fori_loop(
        0, NPOINT, body, (dist0, far0, acc0, acc0, acc0))
    out_ref[:, 0] = ax
    out_ref[:, 1] = ay
    out_ref[:, 2] = az


def _group_kernel(feat_ref, nxyz_ref, w0_ref, c0_ref, z1_ref, st_ref):
    f = feat_ref[0]
    nc = nxyz_ref[0]
    xyz3 = f[0:3]
    sqx = (f[0:1] * f[0:1] + f[1:2] * f[1:2]) + f[2:3] * f[2:3]
    nx = nc[:, 0:1]
    ny = nc[:, 1:2]
    nz = nc[:, 2:3]
    sqs = (nx * nx + ny * ny) + nz * nz
    inner = _dot_bf16(nc, xyz3, (((1,), (0,)), ((), ())))
    sqrd = (sqs + sqx) - 2.0 * inner
    msk = (sqrd <= RADIUS * RADIUS).astype(jnp.float32)

    tri = (jax.lax.broadcasted_iota(jnp.int32, (128, 128), 0)
           <= jax.lax.broadcasted_iota(jnp.int32, (128, 128), 1)
           ).astype(jnp.float32)
    offset = jnp.zeros((SCHUNK, 1), dtype=jnp.float32)
    cparts = []
    for k in range(N // 128):
        mk = msk[:, k * 128:(k + 1) * 128]
        csk = jax.lax.dot_general(mk, tri, (((1,), (0,)), ((), ())),
                                  precision=HIGH)
        cparts.append(csk + offset)
        offset = offset + csk[:, 127:128]
    c = jnp.concatenate(cparts, axis=1)
    cc = msk * c
    count = offset
    ccl = jnp.minimum(cc, 33.0).astype(jnp.bfloat16)
    f_hi = f.astype(jnp.bfloat16)
    f_lo = (f - f_hi.astype(jnp.float32)).astype(jnp.bfloat16)

    w0 = w0_ref[...]
    c0 = c0_ref[...]
    cvec = jnp.concatenate(
        [nc, jnp.zeros((SCHUNK, 3), dtype=jnp.float32)], axis=1)

    s_acc = jnp.zeros((64, 1), dtype=jnp.float32)
    q_acc = jnp.zeros((64, 1), dtype=jnp.float32)
    g_first = None
    dims = (((1,), (1,)), ((), ()))
    for j in range(NSAMPLE):
        match = (ccl == jnp.bfloat16(j + 1)).astype(jnp.bfloat16)
        g = (jax.lax.dot_general(match, f_hi, dims,
                                 preferred_element_type=jnp.float32)
             + jax.lax.dot_general(match, f_lo, dims,
                                   preferred_element_type=jnp.float32))
        gc = g - cvec
        if j == 0:
            g_first = gc
            gfin = gc
        else:
            gfin = jnp.where(count >= float(j + 1), gc, g_first)
        z = _dot_bf16(w0, gfin, (((1,), (1,)), ((), ()))) + c0
        z1_ref[0, j] = z
        s_acc = s_acc + jnp.sum(z, axis=1, keepdims=True)
        q_acc = q_acc + jnp.sum(z * z, axis=1, keepdims=True)

    stacked = jnp.concatenate(
        [s_acc, q_acc, jnp.zeros((64, 6), dtype=jnp.float32)], axis=1)
    prev = jnp.where(pl.program_id(1) == 0,
                     jnp.zeros((64, 8), dtype=jnp.float32), st_ref[0])
    st_ref[0] = prev + stacked


def _mlp_kernel(z_ref, sc_ref, bi_ref, w_ref, cb_ref, out_ref, st_ref, *,
                cout):
    sc = sc_ref[...]
    bi = bi_ref[...]
    w = w_ref[...]
    cb = cb_ref[...]
    s_acc = jnp.zeros((cout, 1), dtype=jnp.float32)
    q_acc = jnp.zeros((cout, 1), dtype=jnp.float32)
    for j in range(NSAMPLE):
        h = jnp.maximum(z_ref[0, j] * sc + bi, 0.0)
        z = _dot_bf16(w, h, (((1,), (0,)), ((), ()))) + cb
        out_ref[0, j] = z
        s_acc = s_acc + jnp.sum(z, axis=1, keepdims=True)
        q_acc = q_acc + jnp.sum(z * z, axis=1, keepdims=True)
    st_ref[0] = jnp.concatenate(
        [s_acc, q_acc, jnp.zeros((cout, 6), dtype=jnp.float32)], axis=1)


def _pool_kernel(z_ref, sc_ref, bi_ref, out_ref):
    sc = sc_ref[...]
    bi = bi_ref[...]
    m = jnp.maximum(z_ref[0, 0] * sc + bi, 0.0)
    for j in range(1, NSAMPLE):
        m = jnp.maximum(m, jnp.maximum(z_ref[0, j] * sc + bi, 0.0))
    out_ref[0] = m


def _bn_coeffs(stats, gamma, beta):
    cnt = float(B * NPOINT * NSAMPLE)
    s = jnp.sum(stats[:, :, 0], axis=0)
    q = jnp.sum(stats[:, :, 1], axis=0)
    mean = s / cnt
    var = q / cnt - mean * mean
    rstd = 1.0 / jnp.sqrt(var + 1e-5)
    scale = gamma * rstd
    bias = beta - mean * scale
    return scale[:, None], bias[:, None]


@jax.jit
def kernel(xyz, points, W0, c0, g0, b0, W1, c1, g1, b1, W2, c2, g2, b2):
    xyz4 = xyz.reshape(B, 3, 32, 128)
    nxyz = pl.pallas_call(
        _fps_kernel,
        out_shape=jax.ShapeDtypeStruct((B, 3, 4, 128), jnp.float32),
    )(xyz4)
    new_xyz_out = nxyz.reshape(B, 3, NPOINT)

    feat = jnp.concatenate([xyz, points], axis=1)
    nct = jnp.transpose(new_xyz_out, (0, 2, 1))
    w0m = W0[:, :, 0, 0]

    z1, st1 = pl.pallas_call(
        _group_kernel,
        grid=(B, NCHUNKS),
        in_specs=[
            pl.BlockSpec((1, 6, N), lambda b, s: (b, 0, 0)),
            pl.BlockSpec((1, SCHUNK, 3), lambda b, s: (b, s, 0)),
            pl.BlockSpec((64, 6), lambda b, s: (0, 0)),
            pl.BlockSpec((64, 1), lambda b, s: (0, 0)),
        ],
        out_specs=[
            pl.BlockSpec((1, NSAMPLE, 64, SCHUNK), lambda b, s: (b, 0, 0, s)),
            pl.BlockSpec((1, 64, 8), lambda b, s: (b, 0, 0)),
        ],
        out_shape=[
            jax.ShapeDtypeStruct((B, NSAMPLE, 64, NPOINT), jnp.float32),
            jax.ShapeDtypeStruct((B, 64, 8), jnp.float32),
        ],
        compiler_params=pltpu.CompilerParams(
            dimension_semantics=("parallel", "arbitrary")),
    )(feat, nct, w0m, c0[:, None])

    sc1, bi1 = _bn_coeffs(st1, g0, b0)
    z2, st2 = pl.pallas_call(
        functools.partial(_mlp_kernel, cout=64),
        grid=(B,),
        in_specs=[
            pl.BlockSpec((1, NSAMPLE, 64, NPOINT), lambda b: (b, 0, 0, 0)),
            pl.BlockSpec((64, 1), lambda b: (0, 0)),
            pl.BlockSpec((64, 1), lambda b: (0, 0)),
            pl.BlockSpec((64, 64), lambda b: (0, 0)),
            pl.BlockSpec((64, 1), lambda b: (0, 0)),
        ],
        out_specs=[
            pl.BlockSpec((1, NSAMPLE, 64, NPOINT), lambda b: (b, 0, 0, 0)),
            pl.BlockSpec((1, 64, 8), lambda b: (b, 0, 0)),
        ],
        out_shape=[
            jax.ShapeDtypeStruct((B, NSAMPLE, 64, NPOINT), jnp.float32),
            jax.ShapeDtypeStruct((B, 64, 8), jnp.float32),
        ],
        compiler_params=pltpu.CompilerParams(
            dimension_semantics=("parallel",)),
    )(z1, sc1, bi1, W1[:, :, 0, 0], c1[:, None])

    sc2, bi2 = _bn_coeffs(st2, g1, b1)
    z3, st3 = pl.pallas_call(
        functools.partial(_mlp_kernel, cout=128),
        grid=(B,),
        in_specs=[
            pl.BlockSpec((1, NSAMPLE, 64, NPOINT), lambda b: (b, 0, 0, 0)),
            pl.BlockSpec((64, 1), lambda b: (0, 0)),
            pl.BlockSpec((64, 1), lambda b: (0, 0)),
            pl.BlockSpec((128, 64), lambda b: (0, 0)),
            pl.BlockSpec((128, 1), lambda b: (0, 0)),
        ],
        out_specs=[
            pl.BlockSpec((1, NSAMPLE, 128, NPOINT), lambda b: (b, 0, 0, 0)),
            pl.BlockSpec((1, 128, 8), lambda b: (b, 0, 0)),
        ],
        out_shape=[
            jax.ShapeDtypeStruct((B, NSAMPLE, 128, NPOINT), jnp.float32),
            jax.ShapeDtypeStruct((B, 128, 8), jnp.float32),
        ],
        compiler_params=pltpu.CompilerParams(
            dimension_semantics=("parallel",)),
    )(z2, sc2, bi2, W2[:, :, 0, 0], c2[:, None])

    sc3, bi3 = _bn_coeffs(st3, g2, b2)
    new_points_out = pl.pallas_call(
        _pool_kernel,
        grid=(B,),
        in_specs=[
            pl.BlockSpec((1, NSAMPLE, 128, NPOINT), lambda b: (b, 0, 0, 0)),
            pl.BlockSpec((128, 1), lambda b: (0, 0)),
            pl.BlockSpec((128, 1), lambda b: (0, 0)),
        ],
        out_specs=pl.BlockSpec((1, 128, NPOINT), lambda b: (b, 0, 0)),
        out_shape=jax.ShapeDtypeStruct((B, 128, NPOINT), jnp.float32),
        compiler_params=pltpu.CompilerParams(
            dimension_semantics=("parallel",)),
    )(z3, sc3, bi3)

    return (new_xyz_out, new_points_out)
